# Initial kernel scaffold; baseline (speedup 1.0000x reference)
import jax
import jax.numpy as jnp
from jax.experimental import pallas as pl


def kernel(x, table):
    raise NotImplementedError("write your pallas kernel here")


# speedup vs baseline: 4.8924x; 4.8924x over previous
import functools

import jax
import jax.numpy as jnp
from jax import lax
from jax.experimental import pallas as pl
from jax.experimental.pallas import tpu as pltpu
from jax.experimental.pallas import tpu_sc as plsc

_B = 4096
_S = 50
_D = 64
_NC = 2
_NS = 16
_NW = _NC * _NS
_BPW = _B // _NW
_LANES = 16
_DCH = _D // _LANES


def _gather_sum_kernel(x_t, table):
  mesh = plsc.VectorSubcoreMesh(core_axis_name="c", subcore_axis_name="s")

  @functools.partial(
      pl.kernel,
      out_type=jax.ShapeDtypeStruct((_B, _D), jnp.float32),
      mesh=mesh,
      compiler_params=pltpu.CompilerParams(use_tc_tiling_on_sc=False),
      scratch_types=[
          pltpu.VMEM((_S, _BPW), jnp.int32),
          pltpu.VMEM((2, _BPW, _D), jnp.float32),
          pltpu.VMEM((_BPW, _D), jnp.float32),
          pltpu.SemaphoreType.DMA,
          pltpu.SemaphoreType.DMA,
      ],
  )
  def k(x_hbm, tbl_hbm, out_hbm, idx_v, rows_v, acc_v, sem0, sem1):
    w = lax.axis_index("s") * _NC + lax.axis_index("c")
    pltpu.sync_copy(x_hbm.at[w], idx_v)

    sems = (sem0, sem1)
    pltpu.async_copy(tbl_hbm.at[idx_v.at[0]], rows_v.at[0], sems[0])

    for s in range(_S):
      p = s % 2
      if s + 1 < _S:
        pn = (s + 1) % 2
        pltpu.async_copy(tbl_hbm.at[idx_v.at[s + 1]], rows_v.at[pn], sems[pn])
      pltpu.make_async_copy(tbl_hbm.at[idx_v.at[s]], rows_v.at[p], sems[p]).wait()

      if s == 0:
        def init_body(j, _):
          for d in range(_DCH):
            sl = pl.ds(d * _LANES, _LANES)
            acc_v[j, sl] = rows_v[0, j, sl]
          return 0
        lax.fori_loop(0, _BPW, init_body, 0, unroll=4)
      else:
        def add_body(j, _, _p=p):
          for d in range(_DCH):
            sl = pl.ds(d * _LANES, _LANES)
            acc_v[j, sl] = acc_v[j, sl] + rows_v[_p, j, sl]
          return 0
        lax.fori_loop(0, _BPW, add_body, 0, unroll=4)

    pltpu.sync_copy(acc_v, out_hbm.at[pl.ds(w * _BPW, _BPW)])

  return k(x_t, table)


def _softmax_scaled(sums):
  blk = 512

  def body(s_ref, o_ref):
    v = s_ref[...] * (1.0 / _S)
    m = jnp.max(v, axis=-1, keepdims=True)
    e = jnp.exp(v - m)
    o_ref[...] = e / jnp.sum(e, axis=-1, keepdims=True)

  return pl.pallas_call(
      body,
      out_shape=jax.ShapeDtypeStruct((_B, _D), jnp.float32),
      grid=(_B // blk,),
      in_specs=[pl.BlockSpec((blk, _D), lambda i: (i, 0))],
      out_specs=pl.BlockSpec((blk, _D), lambda i: (i, 0)),
  )(sums)


def kernel(x, table):
  x = x.astype(jnp.int32)
  x_t = x.reshape(_NW, _BPW, _S).transpose(0, 2, 1)
  sums = _gather_sum_kernel(x_t, table)
  return _softmax_scaled(sums)


# speedup vs baseline: 9.0424x; 1.8483x over previous
import functools

import jax
import jax.numpy as jnp
from jax import lax
from jax.experimental import pallas as pl
from jax.experimental.pallas import tpu as pltpu
from jax.experimental.pallas import tpu_sc as plsc

_B = 4096
_S = 50
_D = 64
_NC = 2
_NS = 16
_NW = _NC * _NS
_BPW = _B // _NW
_LANES = 16
_DCH = _D // _LANES
_K = 5
_G = _S // _K


def _gather_sum_kernel(x_r, table):
  mesh = plsc.VectorSubcoreMesh(core_axis_name="c", subcore_axis_name="s")

  @functools.partial(
      pl.kernel,
      out_type=jax.ShapeDtypeStruct((_B, _D), jnp.float32),
      mesh=mesh,
      compiler_params=pltpu.CompilerParams(
          use_tc_tiling_on_sc=False, needs_layout_passes=False),
      scratch_types=[
          pltpu.VMEM((_BPW * _S,), jnp.int32),
          pltpu.VMEM((_S, _BPW), jnp.int32),
          pltpu.VMEM((2, _K, _BPW, _D), jnp.float32),
          pltpu.VMEM((_BPW, _D), jnp.float32),
          pltpu.SemaphoreType.DMA,
          pltpu.SemaphoreType.DMA,
      ],
  )
  def k(x_hbm, tbl_hbm, out_hbm, xblk_v, idx_v, rows_v, acc_v, sem0, sem1):
    w = lax.axis_index("s") * _NC + lax.axis_index("c")
    pltpu.sync_copy(x_hbm.at[w], xblk_v)
    sems = (sem0, sem1)
    lanes16 = lax.iota(jnp.int32, 16)

    lanes_s = lanes16 * _S

    def transpose_rows(lo, hi):
      def tbody(s, _):
        base = lanes_s + s
        for jb in range(_BPW // _LANES):
          v = plsc.load_gather(xblk_v, [base + (jb * _LANES * _S)])
          idx_v[s, pl.ds(jb * _LANES, _LANES)] = v
        return 0
      lax.fori_loop(lo, hi, tbody, 0)

    def fire(g, p):
      for kk in range(_K):
        s = g * _K + kk
        pltpu.async_copy(tbl_hbm.at[idx_v.at[s]], rows_v.at[p, kk], sems[p])

    def drain(g, p):
      for kk in range(_K):
        s = g * _K + kk
        pltpu.make_async_copy(
            tbl_hbm.at[idx_v.at[s]], rows_v.at[p, kk], sems[p]).wait()

    transpose_rows(0, _K)
    fire(0, 0)
    transpose_rows(_K, _S)

    for g in range(_G):
      p = g % 2
      if g + 1 < _G:
        fire(g + 1, (g + 1) % 2)
      drain(g, p)

      if g == 0:
        def body0(j, _):
          for d in range(_DCH):
            sl = pl.ds(d * _LANES, _LANES)
            a = rows_v[0, 0, j, sl]
            for kk in range(1, _K):
              a = a + rows_v[0, kk, j, sl]
            acc_v[j, sl] = a
          return 0
        lax.fori_loop(0, _BPW, body0, 0, unroll=2)
      else:
        def body(j, _, _p=p):
          for d in range(_DCH):
            sl = pl.ds(d * _LANES, _LANES)
            a = acc_v[j, sl]
            for kk in range(_K):
              a = a + rows_v[_p, kk, j, sl]
            acc_v[j, sl] = a
          return 0
        lax.fori_loop(0, _BPW, body, 0, unroll=2)

    pltpu.sync_copy(acc_v, out_hbm.at[pl.ds(w * _BPW, _BPW)])

  return k(x_r, table)


def _softmax_scaled(sums):
  blk = 512

  def body(s_ref, o_ref):
    v = s_ref[...] * (1.0 / _S)
    m = jnp.max(v, axis=-1, keepdims=True)
    e = jnp.exp(v - m)
    o_ref[...] = e / jnp.sum(e, axis=-1, keepdims=True)

  return pl.pallas_call(
      body,
      out_shape=jax.ShapeDtypeStruct((_B, _D), jnp.float32),
      grid=(_B // blk,),
      in_specs=[pl.BlockSpec((blk, _D), lambda i: (i, 0))],
      out_specs=pl.BlockSpec((blk, _D), lambda i: (i, 0)),
  )(sums)


def kernel(x, table):
  x = x.astype(jnp.int32)
  x_r = x.reshape(_NW, _BPW * _S)
  sums = _gather_sum_kernel(x_r, table)
  return _softmax_scaled(sums)
